# baseline (speedup 1.0000x reference)
import functools

import jax
import jax.numpy as jnp
from jax import lax
from jax.experimental import pallas as pl
from jax.experimental.pallas import tpu as pltpu
from jax.experimental.pallas import tpu_sc as plsc

N = 10000
D = 128
NP = 10240
NW = 32
NSUB = 16
CH = 80
CL = 128
CHD = CH
CLD = CL
EP = NW * CH * CL
RPS = NP // NSUB
ZR = 64


def _sc_mesh():
    return plsc.VectorSubcoreMesh(core_axis_name="c", subcore_axis_name="s")


def _deg_sc(src3, dst3, zeros1):
    @functools.partial(
        pl.kernel,
        out_type=[jax.ShapeDtypeStruct((2, NP), jnp.float32),
                  jax.ShapeDtypeStruct((NW, CHD, CLD), jnp.int32)],
        scratch_types=[pltpu.VMEM((CHD, CLD), jnp.int32),
                       pltpu.VMEM((CHD, CLD), jnp.int32),
                       pltpu.VMEM((CHD, CLD), jnp.float32),
                       pltpu.VMEM_SHARED((NP,), jnp.float32)],
        mesh=_sc_mesh(),
    )
    def k(src_h, dst_h, z_h, degp_h, dstp_h, src_v, dst_v, keep_v, deg_acc):
        c = lax.axis_index("c")
        s = lax.axis_index("s")
        w = s * 2 + c
        pltpu.sync_copy(src_h.at[w], src_v)
        pltpu.sync_copy(dst_h.at[w], dst_v)
        pltpu.sync_copy(z_h.at[pl.ds(s * RPS, RPS)],
                        deg_acc.at[pl.ds(s * RPS, RPS)])
        plsc.subcore_barrier()

        def body(j, carry):
            for c8 in range(CLD // 16):
                sl = pl.ds(c8 * 16, 16)
                sv = src_v[j, sl]
                dv = dst_v[j, sl]
                eq = sv == dv
                keep_v[j, sl] = jnp.where(eq, 0.0, 1.0)
                dst_v[j, sl] = jnp.where(eq, N, dv)
            pltpu.sync_copy(keep_v.at[j], deg_acc.at[src_v.at[j]], add=True)
            return carry

        lax.fori_loop(0, CHD, body, 0)
        plsc.subcore_barrier()
        pltpu.sync_copy(deg_acc.at[pl.ds(s * RPS, RPS)],
                        degp_h.at[c, pl.ds(s * RPS, RPS)])
        pltpu.sync_copy(dst_v, dstp_h.at[w])

    return k(src3, dst3, zeros1)


def _hop_sc(g, src3, dstp3, zeros2):
    @functools.partial(
        pl.kernel,
        out_type=jax.ShapeDtypeStruct((2, NP, D), jnp.float32),
        scratch_types=[pltpu.VMEM((CH, CL), jnp.int32),
                       pltpu.VMEM((CH, CL), jnp.int32),
                       pltpu.VMEM((CL, D), jnp.float32),
                       pltpu.VMEM_SHARED((NP, D), jnp.float32)],
        mesh=_sc_mesh(),
    )
    def k(g_h, src_h, dstp_h, z_h, acc_h, src_v, dst_v, buf, acc):
        c = lax.axis_index("c")
        s = lax.axis_index("s")
        w = s * 2 + c
        pltpu.sync_copy(src_h.at[w], src_v)
        pltpu.sync_copy(dstp_h.at[w], dst_v)
        for z in range(RPS // ZR):
            pltpu.sync_copy(z_h, acc.at[pl.ds(s * RPS + z * ZR, ZR)])
        plsc.subcore_barrier()

        def body(j, carry):
            pltpu.sync_copy(g_h.at[src_v.at[j]], buf)
            pltpu.sync_copy(buf, acc.at[dst_v.at[j]], add=True)
            return carry

        lax.fori_loop(0, CH, body, 0)
        plsc.subcore_barrier()
        pltpu.sync_copy(acc.at[pl.ds(s * RPS, RPS)],
                        acc_h.at[c, pl.ds(s * RPS, RPS)])

    return k(g, src3, dstp3, zeros2)


def _prep_tc(deg_p, x_pad):
    R = 512
    grid = NP // R

    def body(dp_ref, x_ref, dis_ref, g_ref):
        deg = dp_ref[0] + dp_ref[1] + 1.0
        dis = lax.rsqrt(deg)
        dis_ref[...] = dis
        g_ref[...] = x_ref[...] * dis

    return pl.pallas_call(
        body,
        grid=(grid,),
        in_specs=[pl.BlockSpec((2, R, 1), lambda i: (0, i, 0)),
                  pl.BlockSpec((R, D), lambda i: (i, 0))],
        out_specs=[pl.BlockSpec((R, 1), lambda i: (i, 0)),
                   pl.BlockSpec((R, D), lambda i: (i, 0))],
        out_shape=[jax.ShapeDtypeStruct((NP, 1), jnp.float32),
                   jax.ShapeDtypeStruct((NP, D), jnp.float32)],
    )(deg_p, x_pad)


def _mid_tc(accs, g1, dis):
    R = 512
    grid = NP // R

    def body(a_ref, g_ref, dis_ref, h_ref, g2_ref):
        dis_b = dis_ref[...]
        h1 = (a_ref[0] + a_ref[1] + g_ref[...]) * dis_b
        h_ref[...] = h1
        g2_ref[...] = h1 * dis_b

    return pl.pallas_call(
        body,
        grid=(grid,),
        in_specs=[pl.BlockSpec((2, R, D), lambda i: (0, i, 0)),
                  pl.BlockSpec((R, D), lambda i: (i, 0)),
                  pl.BlockSpec((R, 1), lambda i: (i, 0))],
        out_specs=[pl.BlockSpec((R, D), lambda i: (i, 0)),
                   pl.BlockSpec((R, D), lambda i: (i, 0))],
        out_shape=[jax.ShapeDtypeStruct((NP, D), jnp.float32),
                   jax.ShapeDtypeStruct((NP, D), jnp.float32)],
    )(accs, g1, dis)


def _final_tc(accs, g2, dis, x_pad, h1, W, b2, a2):
    R = 400
    grid = N // R

    def body(a_ref, g_ref, dis_ref, x_ref, h1_ref, w_ref, b_ref, s_ref, o_ref):
        h2 = (a_ref[0] + a_ref[1] + g_ref[...]) * dis_ref[...]
        acc = jnp.dot(x_ref[...], w_ref[0:128, :],
                      preferred_element_type=jnp.float32)
        acc = acc + jnp.dot(h1_ref[...], w_ref[128:256, :],
                            preferred_element_type=jnp.float32)
        acc = acc + jnp.dot(h2, w_ref[256:384, :],
                            preferred_element_type=jnp.float32)
        acc = acc + b_ref[...]
        slope = s_ref[0, 0]
        o_ref[...] = jnp.where(acc > 0, acc, slope * acc)

    return pl.pallas_call(
        body,
        grid=(grid,),
        in_specs=[pl.BlockSpec((2, R, D), lambda i: (0, i, 0)),
                  pl.BlockSpec((R, D), lambda i: (i, 0)),
                  pl.BlockSpec((R, 1), lambda i: (i, 0)),
                  pl.BlockSpec((R, D), lambda i: (i, 0)),
                  pl.BlockSpec((R, D), lambda i: (i, 0)),
                  pl.BlockSpec((3 * D, D), lambda i: (0, 0)),
                  pl.BlockSpec((1, D), lambda i: (0, 0)),
                  pl.BlockSpec((1, 1), lambda i: (0, 0))],
        out_specs=pl.BlockSpec((R, D), lambda i: (i, 0)),
        out_shape=jax.ShapeDtypeStruct((N, D), jnp.float32),
    )(accs, g2, dis, x_pad, h1, W, b2, a2)


def kernel(x, edge_index, W, b, a):
    E = edge_index.shape[1]
    pad = EP - E
    src = edge_index[0]
    dst = edge_index[1]
    zpad = jnp.zeros((pad,), jnp.int32)
    src3d = jnp.concatenate([src, zpad]).reshape(NW, CHD, CLD)
    dst3d = jnp.concatenate([dst, zpad]).reshape(NW, CHD, CLD)
    x_pad = jnp.pad(x, ((0, NP - N), (0, 0)))
    zeros1 = jnp.zeros((NP,), jnp.float32)
    zeros2 = jnp.zeros((ZR, D), jnp.float32)

    deg_p, dstp3d = _deg_sc(src3d, dst3d, zeros1)
    src3 = src3d.reshape(NW, CH, CL)
    dstp3 = dstp3d.reshape(NW, CH, CL)
    dis, g1 = _prep_tc(deg_p.reshape(2, NP, 1), x_pad)
    acc1 = _hop_sc(g1, src3, dstp3, zeros2)
    h1, g2 = _mid_tc(acc1, g1, dis)
    acc2 = _hop_sc(g2, src3, dstp3, zeros2)
    out = _final_tc(acc2, g2, dis, x_pad, h1, W,
                    b.reshape(1, D), a.reshape(1, 1))
    return out


# speedup vs baseline: 1.4974x; 1.4974x over previous
import functools

import jax
import jax.numpy as jnp
from jax import lax
from jax.experimental import pallas as pl
from jax.experimental.pallas import tpu as pltpu
from jax.experimental.pallas import tpu_sc as plsc

N = 10000
D = 128
NP = 10240
NW = 32
NSUB = 16
CH = 79
CL = 128
CHD = CH
CLD = CL
EP = NW * CH * CL
RPS = NP // NSUB
ZR = 64


def _sc_mesh():
    return plsc.VectorSubcoreMesh(core_axis_name="c", subcore_axis_name="s")


def _deg_sc(src3, dst3, zeros1):
    @functools.partial(
        pl.kernel,
        out_type=[jax.ShapeDtypeStruct((2, NP), jnp.float32),
                  jax.ShapeDtypeStruct((NW, CHD, CLD), jnp.int32)],
        scratch_types=[pltpu.VMEM((CHD, CLD), jnp.int32),
                       pltpu.VMEM((CHD, CLD), jnp.int32),
                       pltpu.VMEM((CHD, CLD), jnp.float32),
                       pltpu.VMEM_SHARED((NP,), jnp.float32)],
        mesh=_sc_mesh(),
    )
    def k(src_h, dst_h, z_h, degp_h, dstp_h, src_v, dst_v, keep_v, deg_acc):
        c = lax.axis_index("c")
        s = lax.axis_index("s")
        w = s * 2 + c
        pltpu.sync_copy(src_h.at[w], src_v)
        pltpu.sync_copy(dst_h.at[w], dst_v)
        pltpu.sync_copy(z_h.at[pl.ds(s * RPS, RPS)],
                        deg_acc.at[pl.ds(s * RPS, RPS)])
        plsc.subcore_barrier()

        def body(j, carry):
            for c8 in range(CLD // 16):
                sl = pl.ds(c8 * 16, 16)
                sv = src_v[j, sl]
                dv = dst_v[j, sl]
                eq = sv == dv
                keep_v[j, sl] = jnp.where(eq, 0.0, 1.0)
                dst_v[j, sl] = jnp.where(eq, N, dv)
            pltpu.sync_copy(keep_v.at[j], deg_acc.at[src_v.at[j]], add=True)
            return carry

        lax.fori_loop(0, CHD, body, 0)
        plsc.subcore_barrier()
        pltpu.sync_copy(deg_acc.at[pl.ds(s * RPS, RPS)],
                        degp_h.at[c, pl.ds(s * RPS, RPS)])
        pltpu.sync_copy(dst_v, dstp_h.at[w])

    return k(src3, dst3, zeros1)


def _hop_sc(g, src3, dstp3, zeros2):
    @functools.partial(
        pl.kernel,
        out_type=jax.ShapeDtypeStruct((2, NP, D), jnp.float32),
        scratch_types=[pltpu.VMEM((CH, CL), jnp.int32),
                       pltpu.VMEM((CH, CL), jnp.int32),
                       pltpu.VMEM((CL, D), jnp.float32),
                       pltpu.VMEM_SHARED((NP, D), jnp.float32)],
        mesh=_sc_mesh(),
    )
    def k(g_h, src_h, dstp_h, z_h, acc_h, src_v, dst_v, buf, acc):
        c = lax.axis_index("c")
        s = lax.axis_index("s")
        w = s * 2 + c
        pltpu.sync_copy(src_h.at[w], src_v)
        pltpu.sync_copy(dstp_h.at[w], dst_v)
        for z in range(RPS // ZR):
            pltpu.sync_copy(z_h, acc.at[pl.ds(s * RPS + z * ZR, ZR)])
        plsc.subcore_barrier()

        def body(j, carry):
            pltpu.sync_copy(g_h.at[src_v.at[j]], buf)
            pltpu.sync_copy(buf, acc.at[dst_v.at[j]], add=True)
            return carry

        lax.fori_loop(0, CH, body, 0)
        plsc.subcore_barrier()
        pltpu.sync_copy(acc.at[pl.ds(s * RPS, RPS)],
                        acc_h.at[c, pl.ds(s * RPS, RPS)])

    return k(g, src3, dstp3, zeros2)


def _prep_tc(deg_p, x_pad):
    R = 512
    grid = NP // R

    def body(dp_ref, x_ref, dis_ref, g_ref):
        deg = dp_ref[0] + dp_ref[1] + 1.0
        dis = lax.rsqrt(deg)
        dis_ref[...] = dis
        g_ref[...] = x_ref[...] * dis

    return pl.pallas_call(
        body,
        grid=(grid,),
        in_specs=[pl.BlockSpec((2, R, 1), lambda i: (0, i, 0)),
                  pl.BlockSpec((R, D), lambda i: (i, 0))],
        out_specs=[pl.BlockSpec((R, 1), lambda i: (i, 0)),
                   pl.BlockSpec((R, D), lambda i: (i, 0))],
        out_shape=[jax.ShapeDtypeStruct((NP, 1), jnp.float32),
                   jax.ShapeDtypeStruct((NP, D), jnp.float32)],
    )(deg_p, x_pad)


def _mid_tc(accs, g1, dis):
    R = 512
    grid = NP // R

    def body(a_ref, g_ref, dis_ref, h_ref, g2_ref):
        dis_b = dis_ref[...]
        h1 = (a_ref[0] + a_ref[1] + g_ref[...]) * dis_b
        h_ref[...] = h1
        g2_ref[...] = h1 * dis_b

    return pl.pallas_call(
        body,
        grid=(grid,),
        in_specs=[pl.BlockSpec((2, R, D), lambda i: (0, i, 0)),
                  pl.BlockSpec((R, D), lambda i: (i, 0)),
                  pl.BlockSpec((R, 1), lambda i: (i, 0))],
        out_specs=[pl.BlockSpec((R, D), lambda i: (i, 0)),
                   pl.BlockSpec((R, D), lambda i: (i, 0))],
        out_shape=[jax.ShapeDtypeStruct((NP, D), jnp.float32),
                   jax.ShapeDtypeStruct((NP, D), jnp.float32)],
    )(accs, g1, dis)


def _final_tc(accs, g2, dis, x_pad, h1, W, b2, a2):
    R = 400
    grid = N // R

    def body(a_ref, g_ref, dis_ref, x_ref, h1_ref, w_ref, b_ref, s_ref, o_ref):
        h2 = (a_ref[0] + a_ref[1] + g_ref[...]) * dis_ref[...]
        acc = jnp.dot(x_ref[...], w_ref[0:128, :],
                      preferred_element_type=jnp.float32)
        acc = acc + jnp.dot(h1_ref[...], w_ref[128:256, :],
                            preferred_element_type=jnp.float32)
        acc = acc + jnp.dot(h2, w_ref[256:384, :],
                            preferred_element_type=jnp.float32)
        acc = acc + b_ref[...]
        slope = s_ref[0, 0]
        o_ref[...] = jnp.where(acc > 0, acc, slope * acc)

    return pl.pallas_call(
        body,
        grid=(grid,),
        in_specs=[pl.BlockSpec((2, R, D), lambda i: (0, i, 0)),
                  pl.BlockSpec((R, D), lambda i: (i, 0)),
                  pl.BlockSpec((R, 1), lambda i: (i, 0)),
                  pl.BlockSpec((R, D), lambda i: (i, 0)),
                  pl.BlockSpec((R, D), lambda i: (i, 0)),
                  pl.BlockSpec((3 * D, D), lambda i: (0, 0)),
                  pl.BlockSpec((1, D), lambda i: (0, 0)),
                  pl.BlockSpec((1, 1), lambda i: (0, 0))],
        out_specs=pl.BlockSpec((R, D), lambda i: (i, 0)),
        out_shape=jax.ShapeDtypeStruct((N, D), jnp.float32),
    )(accs, g2, dis, x_pad, h1, W, b2, a2)


def kernel(x, edge_index, W, b, a):
    E = edge_index.shape[1]
    pad = EP - E
    src = edge_index[0]
    dst = edge_index[1]
    zpad = jnp.zeros((pad,), jnp.int32)
    src3d = jnp.concatenate([src, zpad]).reshape(NW, CHD, CLD)
    dst3d = jnp.concatenate([dst, zpad]).reshape(NW, CHD, CLD)
    x_pad = jnp.pad(x, ((0, NP - N), (0, 0)))
    zeros1 = jnp.zeros((NP,), jnp.float32)
    zeros2 = jnp.zeros((ZR, D), jnp.float32)

    deg_p, dstp3d = _deg_sc(src3d, dst3d, zeros1)
    src3 = src3d.reshape(NW, CH, CL)
    dstp3 = dstp3d.reshape(NW, CH, CL)
    dis, g1 = _prep_tc(deg_p.reshape(2, NP, 1), x_pad)
    acc1 = _hop_sc(g1, src3, dstp3, zeros2)
    h1, g2 = _mid_tc(acc1, g1, dis)
    acc2 = _hop_sc(g2, src3, dstp3, zeros2)
    out = _final_tc(acc2, g2, dis, x_pad, h1, W,
                    b.reshape(1, D), a.reshape(1, 1))
    return out


# speedup vs baseline: 2.4264x; 1.6204x over previous
import functools

import jax
import jax.numpy as jnp
from jax import lax
from jax.experimental import pallas as pl
from jax.experimental.pallas import tpu as pltpu
from jax.experimental.pallas import tpu_sc as plsc

N = 10000
D = 128
NP = 10240
NW = 32
NSUB = 16
CH = 79
CL = 128
CHD = CH
CLD = CL
EP = NW * CH * CL
RPS = NP // NSUB
ZR = 64


def _sc_mesh():
    return plsc.VectorSubcoreMesh(core_axis_name="c", subcore_axis_name="s")


def _deg_sc(src3, dst3, zeros1):
    @functools.partial(
        pl.kernel,
        out_type=[jax.ShapeDtypeStruct((2, NP), jnp.float32),
                  jax.ShapeDtypeStruct((NW, CHD, CLD), jnp.int32)],
        scratch_types=[pltpu.VMEM((CHD, CLD), jnp.int32),
                       pltpu.VMEM((CHD, CLD), jnp.int32),
                       pltpu.VMEM((CHD, CLD), jnp.float32),
                       pltpu.VMEM_SHARED((NP,), jnp.float32)],
        mesh=_sc_mesh(),
    )
    def k(src_h, dst_h, z_h, degp_h, dstp_h, src_v, dst_v, keep_v, deg_acc):
        c = lax.axis_index("c")
        s = lax.axis_index("s")
        w = s * 2 + c
        pltpu.sync_copy(src_h.at[w], src_v)
        pltpu.sync_copy(dst_h.at[w], dst_v)
        pltpu.sync_copy(z_h.at[pl.ds(s * RPS, RPS)],
                        deg_acc.at[pl.ds(s * RPS, RPS)])
        plsc.subcore_barrier()

        def body(j, carry):
            for c8 in range(CLD // 16):
                sl = pl.ds(c8 * 16, 16)
                sv = src_v[j, sl]
                dv = dst_v[j, sl]
                eq = sv == dv
                keep_v[j, sl] = jnp.where(eq, 0.0, 1.0)
                dst_v[j, sl] = jnp.where(
                    eq, N + jnp.bitwise_and(sv, 127), dv)
            pltpu.sync_copy(keep_v.at[j], deg_acc.at[src_v.at[j]], add=True)
            return carry

        lax.fori_loop(0, CHD, body, 0)
        plsc.subcore_barrier()
        pltpu.sync_copy(deg_acc.at[pl.ds(s * RPS, RPS)],
                        degp_h.at[c, pl.ds(s * RPS, RPS)])
        pltpu.sync_copy(dst_v, dstp_h.at[w])

    return k(src3, dst3, zeros1)


def _hop_sc(g, src3, dstp3, zeros2):
    @functools.partial(
        pl.kernel,
        out_type=jax.ShapeDtypeStruct((2, NP, D), jnp.float32),
        scratch_types=[pltpu.VMEM((CH, CL), jnp.int32),
                       pltpu.VMEM((CH, CL), jnp.int32),
                       pltpu.VMEM((CL, D), jnp.float32),
                       pltpu.VMEM_SHARED((NP, D), jnp.float32)],
        mesh=_sc_mesh(),
    )
    def k(g_h, src_h, dstp_h, z_h, acc_h, src_v, dst_v, buf, acc):
        c = lax.axis_index("c")
        s = lax.axis_index("s")
        w = s * 2 + c
        pltpu.sync_copy(src_h.at[w], src_v)
        pltpu.sync_copy(dstp_h.at[w], dst_v)
        for z in range(RPS // ZR):
            pltpu.sync_copy(z_h, acc.at[pl.ds(s * RPS + z * ZR, ZR)])
        plsc.subcore_barrier()

        def body(j, carry):
            pltpu.sync_copy(g_h.at[src_v.at[j]], buf)
            pltpu.sync_copy(buf, acc.at[dst_v.at[j]], add=True)
            return carry

        lax.fori_loop(0, CH, body, 0)
        plsc.subcore_barrier()
        pltpu.sync_copy(acc.at[pl.ds(s * RPS, RPS)],
                        acc_h.at[c, pl.ds(s * RPS, RPS)])

    return k(g, src3, dstp3, zeros2)


def _prep_tc(deg_p, x_pad):
    R = 512
    grid = NP // R

    def body(dp_ref, x_ref, dis_ref, g_ref):
        deg = dp_ref[0] + dp_ref[1] + 1.0
        dis = lax.rsqrt(deg)
        dis_ref[...] = dis
        g_ref[...] = x_ref[...] * dis

    return pl.pallas_call(
        body,
        grid=(grid,),
        in_specs=[pl.BlockSpec((2, R, 1), lambda i: (0, i, 0)),
                  pl.BlockSpec((R, D), lambda i: (i, 0))],
        out_specs=[pl.BlockSpec((R, 1), lambda i: (i, 0)),
                   pl.BlockSpec((R, D), lambda i: (i, 0))],
        out_shape=[jax.ShapeDtypeStruct((NP, 1), jnp.float32),
                   jax.ShapeDtypeStruct((NP, D), jnp.float32)],
    )(deg_p, x_pad)


def _mid_tc(accs, g1, dis):
    R = 512
    grid = NP // R

    def body(a_ref, g_ref, dis_ref, h_ref, g2_ref):
        dis_b = dis_ref[...]
        h1 = (a_ref[0] + a_ref[1] + g_ref[...]) * dis_b
        h_ref[...] = h1
        g2_ref[...] = h1 * dis_b

    return pl.pallas_call(
        body,
        grid=(grid,),
        in_specs=[pl.BlockSpec((2, R, D), lambda i: (0, i, 0)),
                  pl.BlockSpec((R, D), lambda i: (i, 0)),
                  pl.BlockSpec((R, 1), lambda i: (i, 0))],
        out_specs=[pl.BlockSpec((R, D), lambda i: (i, 0)),
                   pl.BlockSpec((R, D), lambda i: (i, 0))],
        out_shape=[jax.ShapeDtypeStruct((NP, D), jnp.float32),
                   jax.ShapeDtypeStruct((NP, D), jnp.float32)],
    )(accs, g1, dis)


def _final_tc(accs, g2, dis, x_pad, h1, W, b2, a2):
    R = 400
    grid = N // R

    def body(a_ref, g_ref, dis_ref, x_ref, h1_ref, w_ref, b_ref, s_ref, o_ref):
        h2 = (a_ref[0] + a_ref[1] + g_ref[...]) * dis_ref[...]
        acc = jnp.dot(x_ref[...], w_ref[0:128, :],
                      preferred_element_type=jnp.float32)
        acc = acc + jnp.dot(h1_ref[...], w_ref[128:256, :],
                            preferred_element_type=jnp.float32)
        acc = acc + jnp.dot(h2, w_ref[256:384, :],
                            preferred_element_type=jnp.float32)
        acc = acc + b_ref[...]
        slope = s_ref[0, 0]
        o_ref[...] = jnp.where(acc > 0, acc, slope * acc)

    return pl.pallas_call(
        body,
        grid=(grid,),
        in_specs=[pl.BlockSpec((2, R, D), lambda i: (0, i, 0)),
                  pl.BlockSpec((R, D), lambda i: (i, 0)),
                  pl.BlockSpec((R, 1), lambda i: (i, 0)),
                  pl.BlockSpec((R, D), lambda i: (i, 0)),
                  pl.BlockSpec((R, D), lambda i: (i, 0)),
                  pl.BlockSpec((3 * D, D), lambda i: (0, 0)),
                  pl.BlockSpec((1, D), lambda i: (0, 0)),
                  pl.BlockSpec((1, 1), lambda i: (0, 0))],
        out_specs=pl.BlockSpec((R, D), lambda i: (i, 0)),
        out_shape=jax.ShapeDtypeStruct((N, D), jnp.float32),
    )(accs, g2, dis, x_pad, h1, W, b2, a2)


def kernel(x, edge_index, W, b, a):
    E = edge_index.shape[1]
    pad = EP - E
    src = edge_index[0]
    dst = edge_index[1]
    ppw = pad // NW
    pvals = (jnp.arange(pad, dtype=jnp.int32) % N).reshape(NW, ppw)
    src3d = jnp.concatenate(
        [src.reshape(NW, E // NW), pvals], axis=1).reshape(NW, CHD, CLD)
    dst3d = jnp.concatenate(
        [dst.reshape(NW, E // NW), pvals], axis=1).reshape(NW, CHD, CLD)
    x_pad = jnp.pad(x, ((0, NP - N), (0, 0)))
    zeros1 = jnp.zeros((NP,), jnp.float32)
    zeros2 = jnp.zeros((ZR, D), jnp.float32)

    deg_p, dstp3d = _deg_sc(src3d, dst3d, zeros1)
    src3 = src3d.reshape(NW, CH, CL)
    dstp3 = dstp3d.reshape(NW, CH, CL)
    dis, g1 = _prep_tc(deg_p.reshape(2, NP, 1), x_pad)
    acc1 = _hop_sc(g1, src3, dstp3, zeros2)
    h1, g2 = _mid_tc(acc1, g1, dis)
    acc2 = _hop_sc(g2, src3, dstp3, zeros2)
    out = _final_tc(acc2, g2, dis, x_pad, h1, W,
                    b.reshape(1, D), a.reshape(1, 1))
    return out


# speedup vs baseline: 3.3572x; 1.3836x over previous
import functools

import jax
import jax.numpy as jnp
from jax import lax
from jax.experimental import pallas as pl
from jax.experimental.pallas import tpu as pltpu
from jax.experimental.pallas import tpu_sc as plsc

N = 10000
D = 128
NP = 10240
NW = 32
NSUB = 16
CH = 79
CL = 128
CHD = CH
CLD = CL
EP = NW * CH * CL
RPS = NP // NSUB
ZR = 64


def _sc_mesh():
    return plsc.VectorSubcoreMesh(core_axis_name="c", subcore_axis_name="s")


def _deg_sc(src3, dst3, zeros1):
    @functools.partial(
        pl.kernel,
        out_type=[jax.ShapeDtypeStruct((2, NP), jnp.float32),
                  jax.ShapeDtypeStruct((NW, CHD, CLD), jnp.int32)],
        scratch_types=[pltpu.VMEM((CHD, CLD), jnp.int32),
                       pltpu.VMEM((CHD, CLD), jnp.int32),
                       pltpu.VMEM((CHD, CLD), jnp.float32),
                       pltpu.VMEM_SHARED((NP,), jnp.float32)],
        mesh=_sc_mesh(),
    )
    def k(src_h, dst_h, z_h, degp_h, dstp_h, src_v, dst_v, keep_v, deg_acc):
        c = lax.axis_index("c")
        s = lax.axis_index("s")
        w = s * 2 + c
        pltpu.sync_copy(src_h.at[w], src_v)
        pltpu.sync_copy(dst_h.at[w], dst_v)
        pltpu.sync_copy(z_h.at[pl.ds(s * RPS, RPS)],
                        deg_acc.at[pl.ds(s * RPS, RPS)])
        plsc.subcore_barrier()

        def body(j, carry):
            for c8 in range(CLD // 16):
                sl = pl.ds(c8 * 16, 16)
                sv = src_v[j, sl]
                dv = dst_v[j, sl]
                eq = sv == dv
                keep_v[j, sl] = jnp.where(eq, 0.0, 1.0)
                dstp = jnp.where(eq, N + jnp.bitwise_and(sv, 127), dv)
                dst_v[j, sl] = jnp.bitwise_or(sv, lax.shift_left(dstp, 16))
            pltpu.sync_copy(keep_v.at[j], deg_acc.at[src_v.at[j]], add=True)
            return carry

        lax.fori_loop(0, CHD, body, 0)
        plsc.subcore_barrier()
        pltpu.sync_copy(deg_acc.at[pl.ds(s * RPS, RPS)],
                        degp_h.at[c, pl.ds(s * RPS, RPS)])
        pltpu.sync_copy(dst_v, dstp_h.at[w])

    return k(src3, dst3, zeros1)


def _hop_sc(g, pk3, zeros2):
    @functools.partial(
        pl.kernel,
        out_type=jax.ShapeDtypeStruct((2, NP, D), jnp.float32),
        scratch_types=[pltpu.VMEM((CH, CL), jnp.int32),
                       pltpu.VMEM((2, CL), jnp.int32),
                       pltpu.VMEM((CL,), jnp.int32),
                       pltpu.VMEM((CL, D), jnp.float32),
                       pltpu.VMEM((CL, D), jnp.float32),
                       pltpu.VMEM_SHARED((NP, D), jnp.float32),
                       pltpu.SemaphoreType.DMA,
                       pltpu.SemaphoreType.DMA],
        mesh=_sc_mesh(),
    )
    def k(g_h, pk_h, z_h, acc_h, pk_v, sidx, didx, buf0, buf1, acc,
          sem0, sem1):
        c = lax.axis_index("c")
        s = lax.axis_index("s")
        w = s * 2 + c
        pltpu.sync_copy(pk_h.at[w], pk_v)

        def unpack_src(j, slot):
            for c8 in range(CL // 16):
                sl = pl.ds(c8 * 16, 16)
                sidx[slot, sl] = jnp.bitwise_and(pk_v[j, sl], 0xFFFF)

        def unpack_dst(j):
            for c8 in range(CL // 16):
                sl = pl.ds(c8 * 16, 16)
                didx[sl] = lax.shift_right_logical(pk_v[j, sl], 16)

        unpack_src(0, 0)
        pltpu.async_copy(g_h.at[sidx.at[0]], buf0, sem0)
        unpack_src(1, 1)
        pltpu.async_copy(g_h.at[sidx.at[1]], buf1, sem1)
        for z in range(RPS // ZR):
            pltpu.sync_copy(z_h, acc.at[pl.ds(s * RPS + z * ZR, ZR)])
        plsc.subcore_barrier()

        def body(i, carry):
            j0 = 2 * i
            pltpu.make_async_copy(g_h.at[sidx.at[0]], buf0, sem0).wait()
            unpack_dst(j0)
            pltpu.sync_copy(buf0, acc.at[didx], add=True)
            unpack_src(j0 + 2, 0)
            pltpu.async_copy(g_h.at[sidx.at[0]], buf0, sem0)
            j1 = j0 + 1
            pltpu.make_async_copy(g_h.at[sidx.at[1]], buf1, sem1).wait()
            unpack_dst(j1)
            pltpu.sync_copy(buf1, acc.at[didx], add=True)
            unpack_src(j1 + 2, 1)
            pltpu.async_copy(g_h.at[sidx.at[1]], buf1, sem1)
            return carry

        lax.fori_loop(0, (CH - 1) // 2 - 1, body, 0)
        pltpu.make_async_copy(g_h.at[sidx.at[0]], buf0, sem0).wait()
        unpack_dst(CH - 3)
        pltpu.sync_copy(buf0, acc.at[didx], add=True)
        pltpu.make_async_copy(g_h.at[sidx.at[1]], buf1, sem1).wait()
        unpack_dst(CH - 2)
        pltpu.sync_copy(buf1, acc.at[didx], add=True)
        unpack_src(CH - 1, 0)
        pltpu.sync_copy(g_h.at[sidx.at[0]], buf0)
        unpack_dst(CH - 1)
        pltpu.sync_copy(buf0, acc.at[didx], add=True)

        plsc.subcore_barrier()
        pltpu.sync_copy(acc.at[pl.ds(s * RPS, RPS)],
                        acc_h.at[c, pl.ds(s * RPS, RPS)])

    return k(g, pk3, zeros2)


def _prep_tc(deg_p, x_pad):
    R = 512
    grid = NP // R

    def body(dp_ref, x_ref, dis_ref, g_ref):
        deg = dp_ref[0] + dp_ref[1] + 1.0
        dis = lax.rsqrt(deg)
        dis_ref[...] = dis
        g_ref[...] = x_ref[...] * dis

    return pl.pallas_call(
        body,
        grid=(grid,),
        in_specs=[pl.BlockSpec((2, R, 1), lambda i: (0, i, 0)),
                  pl.BlockSpec((R, D), lambda i: (i, 0))],
        out_specs=[pl.BlockSpec((R, 1), lambda i: (i, 0)),
                   pl.BlockSpec((R, D), lambda i: (i, 0))],
        out_shape=[jax.ShapeDtypeStruct((NP, 1), jnp.float32),
                   jax.ShapeDtypeStruct((NP, D), jnp.float32)],
    )(deg_p, x_pad)


def _mid_tc(accs, g1, dis):
    R = 512
    grid = NP // R

    def body(a_ref, g_ref, dis_ref, h_ref, g2_ref):
        dis_b = dis_ref[...]
        h1 = (a_ref[0] + a_ref[1] + g_ref[...]) * dis_b
        h_ref[...] = h1
        g2_ref[...] = h1 * dis_b

    return pl.pallas_call(
        body,
        grid=(grid,),
        in_specs=[pl.BlockSpec((2, R, D), lambda i: (0, i, 0)),
                  pl.BlockSpec((R, D), lambda i: (i, 0)),
                  pl.BlockSpec((R, 1), lambda i: (i, 0))],
        out_specs=[pl.BlockSpec((R, D), lambda i: (i, 0)),
                   pl.BlockSpec((R, D), lambda i: (i, 0))],
        out_shape=[jax.ShapeDtypeStruct((NP, D), jnp.float32),
                   jax.ShapeDtypeStruct((NP, D), jnp.float32)],
    )(accs, g1, dis)


def _final_tc(accs, g2, dis, x_pad, h1, W, b2, a2):
    R = 400
    grid = N // R

    def body(a_ref, g_ref, dis_ref, x_ref, h1_ref, w_ref, b_ref, s_ref, o_ref):
        h2 = (a_ref[0] + a_ref[1] + g_ref[...]) * dis_ref[...]
        acc = jnp.dot(x_ref[...], w_ref[0:128, :],
                      preferred_element_type=jnp.float32)
        acc = acc + jnp.dot(h1_ref[...], w_ref[128:256, :],
                            preferred_element_type=jnp.float32)
        acc = acc + jnp.dot(h2, w_ref[256:384, :],
                            preferred_element_type=jnp.float32)
        acc = acc + b_ref[...]
        slope = s_ref[0, 0]
        o_ref[...] = jnp.where(acc > 0, acc, slope * acc)

    return pl.pallas_call(
        body,
        grid=(grid,),
        in_specs=[pl.BlockSpec((2, R, D), lambda i: (0, i, 0)),
                  pl.BlockSpec((R, D), lambda i: (i, 0)),
                  pl.BlockSpec((R, 1), lambda i: (i, 0)),
                  pl.BlockSpec((R, D), lambda i: (i, 0)),
                  pl.BlockSpec((R, D), lambda i: (i, 0)),
                  pl.BlockSpec((3 * D, D), lambda i: (0, 0)),
                  pl.BlockSpec((1, D), lambda i: (0, 0)),
                  pl.BlockSpec((1, 1), lambda i: (0, 0))],
        out_specs=pl.BlockSpec((R, D), lambda i: (i, 0)),
        out_shape=jax.ShapeDtypeStruct((N, D), jnp.float32),
    )(accs, g2, dis, x_pad, h1, W, b2, a2)


def kernel(x, edge_index, W, b, a):
    E = edge_index.shape[1]
    pad = EP - E
    src = edge_index[0]
    dst = edge_index[1]
    ppw = pad // NW
    pvals = (jnp.arange(pad, dtype=jnp.int32) % N).reshape(NW, ppw)
    src3d = jnp.concatenate(
        [src.reshape(NW, E // NW), pvals], axis=1).reshape(NW, CHD, CLD)
    dst3d = jnp.concatenate(
        [dst.reshape(NW, E // NW), pvals], axis=1).reshape(NW, CHD, CLD)
    x_pad = jnp.pad(x, ((0, NP - N), (0, 0)))
    zeros1 = jnp.zeros((NP,), jnp.float32)
    zeros2 = jnp.zeros((ZR, D), jnp.float32)

    deg_p, pk3d = _deg_sc(src3d, dst3d, zeros1)
    pk3 = pk3d.reshape(NW, CH, CL)
    dis, g1 = _prep_tc(deg_p.reshape(2, NP, 1), x_pad)
    acc1 = _hop_sc(g1, pk3, zeros2)
    h1, g2 = _mid_tc(acc1, g1, dis)
    acc2 = _hop_sc(g2, pk3, zeros2)
    out = _final_tc(acc2, g2, dis, x_pad, h1, W,
                    b.reshape(1, D), a.reshape(1, 1))
    return out
